# baseline (speedup 1.0000x reference)
import jax
import jax.numpy as jnp
from jax.experimental import pallas as pl
import jax.experimental.pallas.tpu as pltpu


_BAND = 128
_SLOTS = 12
_H = _BAND // 2


def _diag_squares(r0, c0, m):
    if m <= 128:
        return [(r0, c0, m, m)]
    h = m // 2
    return ([(r0 + h, c0, h, h)]
            + _diag_squares(r0, c0, h)
            + _diag_squares(r0 + h, c0 + h, h))


def _diag_quadrant_copies(x_ref, buf, sem_r, band, slot, k):
    base = k * _BAND
    out = []
    for r0, c0, hr, hc in _diag_squares(0, 0, _BAND):
        out.append(pltpu.make_async_copy(
            x_ref.at[:, pl.ds(band * _BAND + r0, hr),
                     slice(base + c0, base + c0 + hc)],
            buf.at[slot, :, slice(r0, r0 + hr),
                   slice(base + c0, base + c0 + hc)],
            sem_r.at[slot],
        ))
    return out


def _read_band(x_ref, buf, sem_r, band, slot, nc):
    for k in range(nc):
        sl = slice(k * _BAND, (k + 1) * _BAND)

        @pl.when(k < band)
        def _(sl=sl):
            pltpu.make_async_copy(
                x_ref.at[:, pl.ds(band * _BAND, _BAND), sl],
                buf.at[slot, :, :, sl],
                sem_r.at[slot],
            ).start()

        @pl.when(k == band)
        def _(k=k):
            for cp in _diag_quadrant_copies(x_ref, buf, sem_r, band, slot, k):
                cp.start()


def _wait_band(x_ref, buf, sem_r, band, slot, nc):
    for k in range(nc):
        sl = slice(k * _BAND, (k + 1) * _BAND)

        @pl.when(k < band)
        def _(sl=sl):
            pltpu.make_async_copy(
                x_ref.at[:, pl.ds(band * _BAND, _BAND), sl],
                buf.at[slot, :, :, sl],
                sem_r.at[slot],
            ).wait()

        @pl.when(k == band)
        def _(k=k):
            for cp in _diag_quadrant_copies(x_ref, buf, sem_r, band, slot, k):
                cp.wait()


def _write_copy(o_ref, buf, sem_w, band, slot):
    return pltpu.make_async_copy(
        buf.at[slot],
        o_ref.at[:, pl.ds(band * _BAND, _BAND), :],
        sem_w.at[slot],
    )


def _body(x_ref, o_ref, buf, sem_r, sem_w):
    i = pl.program_id(0)
    n = pl.num_programs(0)
    s = x_ref.shape[2]
    nc = s // _BAND
    slot = jax.lax.rem(i, _SLOTS)
    nxt = jax.lax.rem(i + 1, _SLOTS)

    @pl.when(i == 0)
    def _():
        _read_band(x_ref, buf, sem_r, 0, jnp.int32(0), nc)

    @pl.when(jnp.logical_and(i + 1 < n, i >= _SLOTS - 1))
    def _():
        _write_copy(o_ref, buf, sem_w, i - (_SLOTS - 1), nxt).wait()

    @pl.when(i + 1 < n)
    def _():
        _read_band(x_ref, buf, sem_r, i + 1, nxt, nc)

    for k in range(nc):
        sl = slice(k * _BAND, (k + 1) * _BAND)

        @pl.when(k > i)
        def _(sl=sl):
            buf[slot, :, :, sl] = jnp.ones(
                (buf.shape[1], _BAND, _BAND), jnp.float32
            )

    _wait_band(x_ref, buf, sem_r, i, slot, nc)

    for k in range(nc):
        sl = slice(k * _BAND, (k + 1) * _BAND)

        @pl.when(k == i)
        def _(sl=sl):
            r = jax.lax.broadcasted_iota(jnp.int32, (1, _BAND, _BAND), 1)
            c = jax.lax.broadcasted_iota(jnp.int32, (1, _BAND, _BAND), 2)
            buf[slot, :, :, sl] = jnp.where(
                c > r, jnp.float32(1.0), buf[slot, :, :, sl]
            )

    _write_copy(o_ref, buf, sem_w, i, slot).start()

    n_static = x_ref.shape[1] // _BAND

    @pl.when(i == n - 1)
    def _():
        for band in range(max(0, n_static - _SLOTS), n_static):
            _write_copy(o_ref, buf, sem_w, band, band % _SLOTS).wait()


def kernel(x):
    batch, s, _ = x.shape
    n = s // _BAND
    return pl.pallas_call(
        _body,
        grid=(n,),
        in_specs=[pl.BlockSpec(memory_space=pltpu.MemorySpace.HBM)],
        out_specs=pl.BlockSpec(memory_space=pltpu.MemorySpace.HBM),
        out_shape=jax.ShapeDtypeStruct(x.shape, x.dtype),
        scratch_shapes=[
            pltpu.VMEM((_SLOTS, batch, _BAND, s), jnp.float32),
            pltpu.SemaphoreType.DMA((_SLOTS,)),
            pltpu.SemaphoreType.DMA((_SLOTS,)),
        ],
    )(x)


# speedup vs baseline: 1.0348x; 1.0348x over previous
import jax
import jax.numpy as jnp
from jax.experimental import pallas as pl
import jax.experimental.pallas.tpu as pltpu


_BAND = 256
_SLOTS = 7
_H = _BAND // 2


def _diag_squares(r0, c0, m):
    if m <= 128:
        return [(r0, c0, m, m)]
    h = m // 2
    return ([(r0 + h, c0, h, h)]
            + _diag_squares(r0, c0, h)
            + _diag_squares(r0 + h, c0 + h, h))


def _diag_quadrant_copies(x_ref, buf, sem_r, band, slot, k):
    base = k * _BAND
    out = []
    for r0, c0, hr, hc in _diag_squares(0, 0, _BAND):
        out.append(pltpu.make_async_copy(
            x_ref.at[:, pl.ds(band * _BAND + r0, hr),
                     slice(base + c0, base + c0 + hc)],
            buf.at[slot, :, slice(r0, r0 + hr),
                   slice(base + c0, base + c0 + hc)],
            sem_r.at[slot],
        ))
    return out


def _read_band(x_ref, buf, sem_r, band, slot, nc):
    for k in range(nc):
        sl = slice(k * _BAND, (k + 1) * _BAND)

        @pl.when(k < band)
        def _(sl=sl):
            pltpu.make_async_copy(
                x_ref.at[:, pl.ds(band * _BAND, _BAND), sl],
                buf.at[slot, :, :, sl],
                sem_r.at[slot],
            ).start()

        @pl.when(k == band)
        def _(k=k):
            for cp in _diag_quadrant_copies(x_ref, buf, sem_r, band, slot, k):
                cp.start()


def _wait_band(x_ref, buf, sem_r, band, slot, nc):
    for k in range(nc):
        sl = slice(k * _BAND, (k + 1) * _BAND)

        @pl.when(k < band)
        def _(sl=sl):
            pltpu.make_async_copy(
                x_ref.at[:, pl.ds(band * _BAND, _BAND), sl],
                buf.at[slot, :, :, sl],
                sem_r.at[slot],
            ).wait()

        @pl.when(k == band)
        def _(k=k):
            for cp in _diag_quadrant_copies(x_ref, buf, sem_r, band, slot, k):
                cp.wait()


def _write_copy(o_ref, buf, sem_w, band, slot):
    return pltpu.make_async_copy(
        buf.at[slot],
        o_ref.at[:, pl.ds(band * _BAND, _BAND), :],
        sem_w.at[slot],
    )


def _body(x_ref, o_ref, buf, sem_r, sem_w):
    i = pl.program_id(0)
    n = pl.num_programs(0)
    s = x_ref.shape[2]
    nc = s // _BAND
    slot = jax.lax.rem(i, _SLOTS)
    nxt = jax.lax.rem(i + 1, _SLOTS)

    @pl.when(i == 0)
    def _():
        _read_band(x_ref, buf, sem_r, 0, jnp.int32(0), nc)

    @pl.when(jnp.logical_and(i + 1 < n, i >= _SLOTS - 1))
    def _():
        _write_copy(o_ref, buf, sem_w, i - (_SLOTS - 1), nxt).wait()

    @pl.when(i + 1 < n)
    def _():
        _read_band(x_ref, buf, sem_r, i + 1, nxt, nc)

    for k in range(nc):
        sl = slice(k * _BAND, (k + 1) * _BAND)

        @pl.when(k > i)
        def _(sl=sl):
            buf[slot, :, :, sl] = jnp.ones(
                (buf.shape[1], _BAND, _BAND), jnp.float32
            )

    _wait_band(x_ref, buf, sem_r, i, slot, nc)

    for k in range(nc):
        sl = slice(k * _BAND, (k + 1) * _BAND)

        @pl.when(k == i)
        def _(sl=sl):
            r = jax.lax.broadcasted_iota(jnp.int32, (1, _BAND, _BAND), 1)
            c = jax.lax.broadcasted_iota(jnp.int32, (1, _BAND, _BAND), 2)
            buf[slot, :, :, sl] = jnp.where(
                c > r, jnp.float32(1.0), buf[slot, :, :, sl]
            )

    _write_copy(o_ref, buf, sem_w, i, slot).start()

    n_static = x_ref.shape[1] // _BAND

    @pl.when(i == n - 1)
    def _():
        for band in range(max(0, n_static - _SLOTS), n_static):
            _write_copy(o_ref, buf, sem_w, band, band % _SLOTS).wait()


def kernel(x):
    batch, s, _ = x.shape
    n = s // _BAND
    return pl.pallas_call(
        _body,
        grid=(n,),
        in_specs=[pl.BlockSpec(memory_space=pltpu.MemorySpace.HBM)],
        out_specs=pl.BlockSpec(memory_space=pltpu.MemorySpace.HBM),
        out_shape=jax.ShapeDtypeStruct(x.shape, x.dtype),
        scratch_shapes=[
            pltpu.VMEM((_SLOTS, batch, _BAND, s), jnp.float32),
            pltpu.SemaphoreType.DMA((_SLOTS,)),
            pltpu.SemaphoreType.DMA((_SLOTS,)),
        ],
    )(x)


# speedup vs baseline: 1.0754x; 1.0392x over previous
import jax
import jax.numpy as jnp
from jax.experimental import pallas as pl
import jax.experimental.pallas.tpu as pltpu


_BAND = 256
_SLOTS = 6
_H = _BAND // 2


def _diag_squares(r0, c0, m):
    if m <= 128:
        return [(r0, c0, m, m)]
    h = m // 2
    return ([(r0 + h, c0, h, h)]
            + _diag_squares(r0, c0, h)
            + _diag_squares(r0 + h, c0 + h, h))


def _diag_quadrant_copies(x_ref, buf, sem_r, band, slot, k):
    base = k * _BAND
    out = []
    for r0, c0, hr, hc in _diag_squares(0, 0, _BAND):
        out.append(pltpu.make_async_copy(
            x_ref.at[:, pl.ds(band * _BAND + r0, hr),
                     slice(base + c0, base + c0 + hc)],
            buf.at[slot, :, slice(r0, r0 + hr),
                   slice(base + c0, base + c0 + hc)],
            sem_r.at[slot],
        ))
    return out


def _read_band(x_ref, buf, sem_r, band, slot, nc):
    for k in range(nc):
        sl = slice(k * _BAND, (k + 1) * _BAND)

        @pl.when(k < band)
        def _(sl=sl):
            pltpu.make_async_copy(
                x_ref.at[:, pl.ds(band * _BAND, _BAND), sl],
                buf.at[slot, :, :, sl],
                sem_r.at[slot],
            ).start()

        @pl.when(k == band)
        def _(k=k):
            for cp in _diag_quadrant_copies(x_ref, buf, sem_r, band, slot, k):
                cp.start()


def _wait_band(x_ref, buf, sem_r, band, slot, nc):
    for k in range(nc):
        sl = slice(k * _BAND, (k + 1) * _BAND)

        @pl.when(k < band)
        def _(sl=sl):
            pltpu.make_async_copy(
                x_ref.at[:, pl.ds(band * _BAND, _BAND), sl],
                buf.at[slot, :, :, sl],
                sem_r.at[slot],
            ).wait()

        @pl.when(k == band)
        def _(k=k):
            for cp in _diag_quadrant_copies(x_ref, buf, sem_r, band, slot, k):
                cp.wait()


def _write_copy(o_ref, buf, sem_w, band, slot):
    return pltpu.make_async_copy(
        buf.at[slot],
        o_ref.at[:, pl.ds(band * _BAND, _BAND), :],
        sem_w.at[slot],
    )


def _body(x_ref, o_ref, buf, sem_r, sem_w):
    i = pl.program_id(0)
    n = pl.num_programs(0)
    s = x_ref.shape[2]
    nc = s // _BAND
    slot = jax.lax.rem(i, _SLOTS)
    nxt = jax.lax.rem(i + 1, _SLOTS)
    band = n - 1 - i

    @pl.when(i == 0)
    def _():
        _read_band(x_ref, buf, sem_r, band, jnp.int32(0), nc)

    @pl.when(jnp.logical_and(i + 1 < n, i >= _SLOTS - 1))
    def _():
        _write_copy(o_ref, buf, sem_w, band + _SLOTS - 1, nxt).wait()

    @pl.when(i + 1 < n)
    def _():
        _read_band(x_ref, buf, sem_r, band - 1, nxt, nc)

    for k in range(nc):
        sl = slice(k * _BAND, (k + 1) * _BAND)

        @pl.when(k > band)
        def _(sl=sl):
            buf[slot, :, :, sl] = jnp.ones(
                (buf.shape[1], _BAND, _BAND), jnp.float32
            )

    _wait_band(x_ref, buf, sem_r, band, slot, nc)

    for k in range(nc):
        sl = slice(k * _BAND, (k + 1) * _BAND)

        @pl.when(k == band)
        def _(sl=sl):
            r = jax.lax.broadcasted_iota(jnp.int32, (1, _BAND, _BAND), 1)
            c = jax.lax.broadcasted_iota(jnp.int32, (1, _BAND, _BAND), 2)
            buf[slot, :, :, sl] = jnp.where(
                c > r, jnp.float32(1.0), buf[slot, :, :, sl]
            )

    _write_copy(o_ref, buf, sem_w, band, slot).start()

    n_static = x_ref.shape[1] // _BAND

    @pl.when(i == n - 1)
    def _():
        for b in range(min(_SLOTS, n_static)):
            _write_copy(o_ref, buf, sem_w, b, (n_static - 1 - b) % _SLOTS).wait()


def kernel(x):
    batch, s, _ = x.shape
    n = s // _BAND
    return pl.pallas_call(
        _body,
        grid=(n,),
        in_specs=[pl.BlockSpec(memory_space=pltpu.MemorySpace.HBM)],
        out_specs=pl.BlockSpec(memory_space=pltpu.MemorySpace.HBM),
        out_shape=jax.ShapeDtypeStruct(x.shape, x.dtype),
        scratch_shapes=[
            pltpu.VMEM((_SLOTS, batch, _BAND, s), jnp.float32),
            pltpu.SemaphoreType.DMA((_SLOTS,)),
            pltpu.SemaphoreType.DMA((_SLOTS,)),
        ],
    )(x)
